# Initial kernel scaffold; baseline (speedup 1.0000x reference)
import jax
import jax.numpy as jnp
from jax.experimental import pallas as pl


def kernel(x, edge_index, batch, params):
    raise NotImplementedError("write your pallas kernel here")


# speedup vs baseline: 5.3916x; 5.3916x over previous
import functools

import jax
import jax.numpy as jnp
from jax import lax
from jax.experimental import pallas as pl
from jax.experimental.pallas import tpu as pltpu
from jax.experimental.pallas import tpu_sc as plsc

_N = 10000
_E = 320000
_D = 128
_NG = 64
_LAYERS = 8

_NC = 2
_NS = 16
_NW = _NC * _NS
_EPW = _E // _NW
_ECHUNK = 128
_EFULL = _EPW // _ECHUNK
_ETAIL = _EPW - _EFULL * _ECHUNK
_RPT = 632
_RLAST = _N - (_NS - 1) * _RPT
_NPW = 320
_PCHUNK = 80

_mesh = plsc.VectorSubcoreMesh(core_axis_name="c", subcore_axis_name="s")


def _acc_zero(zeros, acc, s):
    @pl.when(s < _NS - 1)
    def _():
        pltpu.sync_copy(zeros.at[pl.ds(s * _RPT, _RPT)],
                        acc.at[pl.ds(s * _RPT, _RPT)])

    @pl.when(s == _NS - 1)
    def _():
        pltpu.sync_copy(zeros.at[pl.ds((_NS - 1) * _RPT, _RLAST)],
                        acc.at[pl.ds((_NS - 1) * _RPT, _RLAST)])


def _acc_writeback(acc, out, c, s):
    @pl.when(s < _NS - 1)
    def _():
        pltpu.sync_copy(acc.at[pl.ds(s * _RPT, _RPT)],
                        out.at[c, pl.ds(s * _RPT, _RPT)])

    @pl.when(s == _NS - 1)
    def _():
        pltpu.sync_copy(acc.at[pl.ds((_NS - 1) * _RPT, _RLAST)],
                        out.at[c, pl.ds((_NS - 1) * _RPT, _RLAST)])


def _sc_segsum_body(h, src, dst, zeros, out,
                    idx_s, idx_d, rows, idx_st, idx_dt, rows_t, acc, sem):
    c = lax.axis_index("c")
    s = lax.axis_index("s")
    wid = c * _NS + s
    _acc_zero(zeros, acc, s)
    plsc.subcore_barrier()
    base = wid * _EPW

    def chunk(t, carry):
        off = base + t * _ECHUNK
        pltpu.sync_copy(src.at[pl.ds(off, _ECHUNK)], idx_s)
        pltpu.sync_copy(dst.at[pl.ds(off, _ECHUNK)], idx_d)
        pltpu.async_copy(h.at[idx_s], rows, sem).wait()
        pltpu.sync_copy(rows, acc.at[idx_d], add=True)
        return carry

    lax.fori_loop(0, _EFULL, chunk, 0)
    off = base + _EFULL * _ECHUNK
    pltpu.sync_copy(src.at[pl.ds(off, _ETAIL)], idx_st)
    pltpu.sync_copy(dst.at[pl.ds(off, _ETAIL)], idx_dt)
    pltpu.async_copy(h.at[idx_st], rows_t, sem).wait()
    pltpu.sync_copy(rows_t, acc.at[idx_dt], add=True)

    plsc.subcore_barrier()
    _acc_writeback(acc, out, c, s)


_sc_segsum = pl.kernel(
    _sc_segsum_body,
    out_type=jax.ShapeDtypeStruct((_NC, _N, _D), jnp.float32),
    mesh=_mesh,
    scratch_types=[
        pltpu.VMEM((_ECHUNK,), jnp.int32),
        pltpu.VMEM((_ECHUNK,), jnp.int32),
        pltpu.VMEM((_ECHUNK, _D), jnp.float32),
        pltpu.VMEM((_ETAIL,), jnp.int32),
        pltpu.VMEM((_ETAIL,), jnp.int32),
        pltpu.VMEM((_ETAIL, _D), jnp.float32),
        pltpu.VMEM_SHARED((_N, _D), jnp.float32),
        pltpu.SemaphoreType.DMA,
    ],
)


def _sc_deg_body(dst, zeros, ones2, out, idx_d, idx_dt, ones_v, acc, sem):
    del sem
    c = lax.axis_index("c")
    s = lax.axis_index("s")
    wid = c * _NS + s
    _acc_zero(zeros, acc, s)
    pltpu.sync_copy(ones2, ones_v)
    plsc.subcore_barrier()
    base = wid * _EPW

    def chunk(t, carry):
        off = base + t * _ECHUNK
        pltpu.sync_copy(dst.at[pl.ds(off, _ECHUNK)], idx_d)
        pltpu.sync_copy(ones_v, acc.at[idx_d], add=True)
        return carry

    lax.fori_loop(0, _EFULL, chunk, 0)
    off = base + _EFULL * _ECHUNK
    pltpu.sync_copy(dst.at[pl.ds(off, _ETAIL)], idx_dt)
    pltpu.sync_copy(ones_v.at[pl.ds(0, _ETAIL)], acc.at[idx_dt], add=True)

    plsc.subcore_barrier()
    _acc_writeback(acc, out, c, s)


_sc_deg = pl.kernel(
    _sc_deg_body,
    out_type=jax.ShapeDtypeStruct((_NC, _N, _D), jnp.float32),
    mesh=_mesh,
    scratch_types=[
        pltpu.VMEM((_ECHUNK,), jnp.int32),
        pltpu.VMEM((_ETAIL,), jnp.int32),
        pltpu.VMEM((_ECHUNK, _D), jnp.float32),
        pltpu.VMEM_SHARED((_N, _D), jnp.float32),
        pltpu.SemaphoreType.DMA,
    ],
)


def _sc_pool_body(xf, batch, zeros, ones2, ssum, cnt,
                  idx_b, rows, ones_v, sacc, cacc, sem):
    del sem
    c = lax.axis_index("c")
    s = lax.axis_index("s")
    wid = c * _NS + s

    @pl.when(s == 0)
    def _():
        pltpu.sync_copy(zeros.at[pl.ds(0, _NG)], sacc)
        pltpu.sync_copy(zeros.at[pl.ds(0, _NG)], cacc)

    pltpu.sync_copy(ones2.at[pl.ds(0, _PCHUNK)], ones_v)
    plsc.subcore_barrier()
    base = wid * _NPW

    def chunk(t, carry):
        off = base + t * _PCHUNK

        @pl.when(off < _N)
        def _():
            pltpu.sync_copy(batch.at[pl.ds(off, _PCHUNK)], idx_b)
            pltpu.sync_copy(xf.at[pl.ds(off, _PCHUNK)], rows)
            pltpu.sync_copy(rows, sacc.at[idx_b], add=True)
            pltpu.sync_copy(ones_v, cacc.at[idx_b], add=True)

        return carry

    lax.fori_loop(0, _NPW // _PCHUNK, chunk, 0)
    plsc.subcore_barrier()

    @pl.when(s == 0)
    def _():
        pltpu.sync_copy(sacc, ssum.at[c])
        pltpu.sync_copy(cacc, cnt.at[c])


_sc_pool = pl.kernel(
    _sc_pool_body,
    out_type=[
        jax.ShapeDtypeStruct((_NC, _NG, _D), jnp.float32),
        jax.ShapeDtypeStruct((_NC, _NG, _D), jnp.float32),
    ],
    mesh=_mesh,
    scratch_types=[
        pltpu.VMEM((_PCHUNK,), jnp.int32),
        pltpu.VMEM((_PCHUNK, _D), jnp.float32),
        pltpu.VMEM((_PCHUNK, _D), jnp.float32),
        pltpu.VMEM_SHARED((_NG, _D), jnp.float32),
        pltpu.VMEM_SHARED((_NG, _D), jnp.float32),
        pltpu.SemaphoreType.DMA,
    ],
)


def _ln_mm(x, g, b, wl, wr):
    mu = jnp.mean(x, axis=-1, keepdims=True)
    xc = x - mu
    var = jnp.mean(xc * xc, axis=-1, keepdims=True)
    ln = xc * lax.rsqrt(var + 1e-5) * g + b
    return (jnp.dot(ln, wl, preferred_element_type=jnp.float32),
            jnp.dot(ln, wr, preferred_element_type=jnp.float32))


def _tc_pre_body(x, g, b, wl, wr, xl_o, hr_o):
    xl, hr = _ln_mm(x[...], g[...], b[...], wl[...], wr[...])
    xl_o[...] = xl
    hr_o[...] = hr


_tc_pre = pl.pallas_call(
    _tc_pre_body,
    out_shape=(
        jax.ShapeDtypeStruct((_N, _D), jnp.float32),
        jax.ShapeDtypeStruct((_N, _D), jnp.float32),
    ),
)


def _tc_mid_body(first, save_x, has_res, *refs):
    (xl, p, dinv, bvec, g2, b2, wl2, wr2), rest = refs[:8], refs[8:]
    if has_res:
        res, rest = rest[0], rest[1:]
    if first:
        invd = 1.0 / jnp.maximum(dinv[0] + dinv[1], 1.0)
        invd_o, rest = rest[0], rest[1:]
        invd_o[...] = invd
    else:
        invd = dinv[...]
    xn = jnp.maximum(xl[...] + (p[0] + p[1]) * invd + bvec[...], 0.0)
    if has_res:
        xn = xn + res[...]
    if save_x:
        x_o, rest = rest[0], rest[1:]
        x_o[...] = xn
    xl_o, hr_o = rest
    xl2, hr2 = _ln_mm(xn, g2[...], b2[...], wl2[...], wr2[...])
    xl_o[...] = xl2
    hr_o[...] = hr2


def _make_mid(first, save_x, has_res):
    shapes = []
    if first:
        shapes.append(jax.ShapeDtypeStruct((_N, _D), jnp.float32))
    if save_x:
        shapes.append(jax.ShapeDtypeStruct((_N, _D), jnp.float32))
    shapes.append(jax.ShapeDtypeStruct((_N, _D), jnp.float32))
    shapes.append(jax.ShapeDtypeStruct((_N, _D), jnp.float32))
    return pl.pallas_call(
        functools.partial(_tc_mid_body, first, save_x, has_res),
        out_shape=tuple(shapes),
    )


_tc_mid_first = _make_mid(True, False, False)
_tc_mid_plain = _make_mid(False, False, False)
_tc_mid_save = _make_mid(False, True, False)
_tc_mid_res = _make_mid(False, False, True)


def _tc_last_body(xl, p, invd, bvec, x_o):
    x_o[...] = jnp.maximum(xl[...] + (p[0] + p[1]) * invd[...] + bvec[...], 0.0)


_tc_last = pl.pallas_call(
    _tc_last_body,
    out_shape=jax.ShapeDtypeStruct((_N, _D), jnp.float32),
)


def _tc_out_body(sp, cp, w, b, o):
    pooled = (sp[0] + sp[1]) / jnp.maximum(cp[0] + cp[1], 1.0)
    o[...] = jnp.dot(pooled, w[...], preferred_element_type=jnp.float32) + b[...]


_tc_out = pl.pallas_call(
    _tc_out_body,
    out_shape=jax.ShapeDtypeStruct((_NG, _D), jnp.float32),
)


def kernel(x, edge_index, batch, params):
    src = edge_index[0]
    dst = edge_index[1]
    zeros = jnp.zeros((_N, _D), jnp.float32)
    ones2 = jnp.ones((_ECHUNK, _D), jnp.float32)

    degb = _sc_deg(dst, zeros, ones2)
    xl, hr = _tc_pre(x, params["gamma0"], params["beta0"],
                     params["Wl0"], params["Wr0"])
    invd = None
    res4 = None
    for i in range(_LAYERS - 1):
        p = _sc_segsum(hr, src, dst, zeros)
        nxt = (params[f"gamma{i + 1}"], params[f"beta{i + 1}"],
               params[f"Wl{i + 1}"], params[f"Wr{i + 1}"])
        if i == 0:
            invd, xl, hr = _tc_mid_first(xl, p, degb, params["b0"], *nxt)
        elif i == 4:
            res4, xl, hr = _tc_mid_save(xl, p, invd, params[f"b{i}"], *nxt)
        elif i == 6:
            xl, hr = _tc_mid_res(xl, p, invd, params[f"b{i}"], *nxt, res4)
        else:
            xl, hr = _tc_mid_plain(xl, p, invd, params[f"b{i}"], *nxt)
    p = _sc_segsum(hr, src, dst, zeros)
    xf = _tc_last(xl, p, invd, params[f"b{_LAYERS - 1}"])
    ssum, cnt = _sc_pool(xf, batch, zeros, ones2)
    return _tc_out(ssum, cnt, params["W_out"], params["b_out"])
